# baseline (speedup 1.0000x reference)
import functools

import jax
import jax.numpy as jnp
from jax import lax
from jax.experimental import pallas as pl
from jax.experimental.pallas import tpu as pltpu
from jax.experimental.pallas import tpu_sc as plsc

_N = 10000
_E = 320000
_D = 128
_H = 64
_E2 = _E // 2

_CHUNK = 128
_NW = 32
_NSEG = 5
_SEG_ROWS = _E2 // _NSEG
_SEG_CHUNKS = _SEG_ROWS // (_CHUNK // 2)
_SEG_ITER = -(-_SEG_CHUNKS // _NW)


def _precompute_body(x_ref, w_ref, a_ref, b_ref):
    ab = jnp.dot(x_ref[...], w_ref[...], preferred_element_type=jnp.float32)
    a_ref[...] = ab[:, :_H]
    b_ref[...] = ab[:, _H:]


def _precompute(x, w1ab):
    return pl.pallas_call(
        _precompute_body,
        out_shape=(jax.ShapeDtypeStruct((_N, _H), jnp.float32),
                   jax.ShapeDtypeStruct((_N, _H), jnp.float32)),
    )(x, w1ab)


@functools.cache
def _make_sc_gather(coff):
    @functools.partial(
        pl.kernel,
        out_type=jax.ShapeDtypeStruct((_SEG_ROWS, 2 * _H), jnp.float32),
        mesh=plsc.VectorSubcoreMesh(core_axis_name="c", subcore_axis_name="s"),
        scratch_types=[
            pltpu.VMEM((2, _CHUNK), jnp.int32),
            pltpu.VMEM((2, _CHUNK), jnp.int32),
            pltpu.VMEM((_CHUNK, _H), jnp.float32),
            pltpu.VMEM((_CHUNK, _H), jnp.float32),
            pltpu.VMEM((_CHUNK, _H), jnp.float32),
            pltpu.VMEM((_CHUNK, _H), jnp.float32),
            pltpu.VMEM((_CHUNK // 2, 2 * _H), jnp.float32),
            pltpu.SemaphoreType.DMA,
            pltpu.SemaphoreType.DMA,
            pltpu.SemaphoreType.DMA,
            pltpu.SemaphoreType.DMA,
        ],
        compiler_params=pltpu.CompilerParams(use_tc_tiling_on_sc=False),
    )
    def _sc_gather(a_hbm, b_hbm, eidx_hbm, out_hbm, idx_s, idx_d,
                   buf_a0, buf_b0, buf_a1, buf_b1, out_buf,
                   sem_a0, sem_b0, sem_a1, sem_b1):
        wid = lax.axis_index("s") * 2 + lax.axis_index("c")
        bufs = ((buf_a0, buf_b0, sem_a0, sem_b0),
                (buf_a1, buf_b1, sem_a1, sem_b1))

        half = _CHUNK // 2

        def fire(j, b):
            c = wid + _NW * j

            @pl.when(c < _SEG_CHUNKS)
            def _():
                base = (coff + c) * half
                pltpu.sync_copy(eidx_hbm.at[0, pl.ds(base, half)],
                                idx_s.at[b, pl.ds(0, half)])
                pltpu.sync_copy(eidx_hbm.at[0, pl.ds(_E2 + base, half)],
                                idx_s.at[b, pl.ds(half, half)])
                pltpu.sync_copy(eidx_hbm.at[1, pl.ds(base, half)],
                                idx_d.at[b, pl.ds(0, half)])
                pltpu.sync_copy(eidx_hbm.at[1, pl.ds(_E2 + base, half)],
                                idx_d.at[b, pl.ds(half, half)])
                buf_a, buf_b, sem_a, sem_b = bufs[b]
                pltpu.async_copy(a_hbm.at[idx_s.at[b]], buf_a, sem_a)
                pltpu.async_copy(b_hbm.at[idx_d.at[b]], buf_b, sem_b)

        def drain_compute(j, b):
            c = wid + _NW * j

            @pl.when(c < _SEG_CHUNKS)
            def _():
                buf_a, buf_b, sem_a, sem_b = bufs[b]
                pltpu.make_async_copy(a_hbm.at[idx_s.at[b]], buf_a,
                                      sem_a).wait()
                pltpu.make_async_copy(b_hbm.at[idx_d.at[b]], buf_b,
                                      sem_b).wait()
                fire(j + 1, 1 - b)

                def add_row(r, carry):
                    for l in range(_H // 16):
                        sl = pl.ds(l * 16, 16)
                        sb = pl.ds(_H + l * 16, 16)
                        out_buf[r, sl] = buf_a[r, sl] + buf_b[r, sl]
                        out_buf[r, sb] = buf_a[half + r, sl] + buf_b[half + r, sl]
                    return carry

                lax.fori_loop(0, half, add_row, 0)
                pltpu.sync_copy(out_buf,
                                out_hbm.at[pl.ds(c * half, half)])

        fire(0, 0)

        def body(j2, carry):
            drain_compute(2 * j2, 0)
            drain_compute(2 * j2 + 1, 1)
            return carry

        lax.fori_loop(0, (_SEG_ITER + 1) // 2, body, 0)

    return _sc_gather


_R2 = 1280


def _mlp_body(h_ref, e_ref, ew_ref, b1_ref, g1_ref, bt1_ref,
              w2_ref, b2_ref, g2_ref, bt2_ref,
              w3_ref, b3_ref, g3_ref, bt3_ref,
              w4_ref, b4_ref, savg_ref, sbc_ref, out_ref):
    savg = savg_ref[...]
    sbc = sbc_ref[...]

    def dot(a, b):
        return jnp.dot(a, b, preferred_element_type=jnp.float32)

    h = (h_ref[...]
         + lax.dot_general(e_ref[...], ew_ref[...], (((0,), (0,)), ((), ())),
                           preferred_element_type=jnp.float32)
         + b1_ref[...])

    def ln_tanh(v, g, bt):
        mu2 = dot(v, savg)
        d = v - dot(mu2, sbc)
        var2 = dot(d * d, savg)
        rstd2 = lax.rsqrt(var2 + 1e-5)
        return jnp.tanh(d * dot(rstd2, sbc) * g + bt)

    h = ln_tanh(h, g1_ref[...], bt1_ref[...])
    h = dot(h, w2_ref[...]) + b2_ref[...]
    h = ln_tanh(h, g2_ref[...], bt2_ref[...])
    h = dot(h, w3_ref[...]) + b3_ref[...]
    h = ln_tanh(h, g3_ref[...], bt3_ref[...])
    out_ref[...] = (lax.dot_general(w4_ref[...], h, (((0,), (1,)), ((), ())),
                                    preferred_element_type=jnp.float32)
                    + b4_ref[...])


def _mlp(hpre, e2, ew, b1, g1, bt1, w2, b2, g2, bt2, w3, b3, g3, bt3,
         w4, b4, savg, sbc):
    nblk = _SEG_ROWS // _R2
    full = lambda shape: pl.BlockSpec(shape, lambda j: (0, 0))
    vec = full((1, 2 * _H))
    return pl.pallas_call(
        _mlp_body,
        grid=(nblk,),
        in_specs=[
            pl.BlockSpec((_R2, 2 * _H), lambda j: (j, 0)),
            pl.BlockSpec((2, _R2), lambda j: (0, j)),
            full((2, 2 * _H)),
            vec, vec, vec,
            full((2 * _H, 2 * _H)), vec, vec, vec,
            full((2 * _H, 2 * _H)), vec, vec, vec,
            full((2 * _H, 2)), full((1, 1)),
            full((2 * _H, 2)), full((2, 2 * _H)),
        ],
        out_specs=pl.BlockSpec((2, _R2), lambda j: (0, j)),
        out_shape=jax.ShapeDtypeStruct((2, _SEG_ROWS), jnp.float32),
        compiler_params=pltpu.CompilerParams(
            dimension_semantics=("arbitrary",)),
    )(hpre, e2, ew, b1, g1, bt1, w2, b2, g2, bt2, w3, b3, g3, bt3, w4, b4,
      savg, sbc)


def _blockdiag2(w):
    a, b = w.shape
    z = jnp.zeros((a, b), w.dtype)
    return jnp.concatenate([jnp.concatenate([w, z], axis=1),
                            jnp.concatenate([z, w], axis=1)], axis=0)


def kernel(x, e, edge_index, W1, b1, W2, b2, W3, b3, W4, b4,
           g1, bt1, g2, bt2, g3, bt3):
    f32 = jnp.float32
    w1ab = jnp.concatenate([W1[:_D], W1[_D:2 * _D]], axis=1)
    a_tab, b_tab = _precompute(x, w1ab)

    w1c = W1[2 * _D]
    zeros_h = jnp.zeros((_H,), f32)
    ew = jnp.stack([jnp.concatenate([w1c, zeros_h]),
                    jnp.concatenate([zeros_h, w1c])])
    ones_h = jnp.ones((_H, 1), f32)
    zeros_col = jnp.zeros((_H, 1), f32)
    savg = jnp.concatenate(
        [jnp.concatenate([ones_h, zeros_col], axis=1),
         jnp.concatenate([zeros_col, ones_h], axis=1)], axis=0) / _H
    sbc = (savg.T > 0).astype(f32) * 1.0

    two = lambda v: jnp.tile(v.reshape(1, _H), (1, 2))
    e2 = e.reshape(2, _E2)
    outs = []
    for k in range(_NSEG):
        hpre_k = _make_sc_gather(k * _SEG_CHUNKS)(a_tab, b_tab, edge_index)
        e2_k = lax.slice(e2, (0, k * _SEG_ROWS), (2, (k + 1) * _SEG_ROWS))
        outs.append(_mlp(hpre_k, e2_k, ew,
                         two(b1), two(g1), two(bt1),
                         _blockdiag2(W2), two(b2), two(g2), two(bt2),
                         _blockdiag2(W3), two(b3), two(g3), two(bt3),
                         _blockdiag2(W4), b4.reshape(1, 1),
                         savg, sbc))
    return jnp.concatenate(outs, axis=1).reshape(_E)


# speedup vs baseline: 1.1845x; 1.1845x over previous
import functools

import jax
import jax.numpy as jnp
from jax import lax
from jax.experimental import pallas as pl
from jax.experimental.pallas import tpu as pltpu
from jax.experimental.pallas import tpu_sc as plsc

_N = 10000
_E = 320000
_D = 128
_H = 64
_E2 = _E // 2

_CHUNK = 128
_NW = 32
_NSEG = 10
_SEG_ROWS = _E2 // _NSEG
_SEG_CHUNKS = _SEG_ROWS // (_CHUNK // 2)
_SEG_ITER = -(-_SEG_CHUNKS // _NW)


def _precompute_body(x_ref, w_ref, ab_ref):
    ab_ref[...] = jnp.dot(x_ref[...], w_ref[...],
                          preferred_element_type=jnp.float32)


def _precompute(x, w1ab):
    return pl.pallas_call(
        _precompute_body,
        out_shape=jax.ShapeDtypeStruct((_N, 2 * _H), jnp.float32),
    )(x, w1ab)


@functools.cache
def _make_sc_gather(coff):
    @functools.partial(
        pl.kernel,
        out_type=jax.ShapeDtypeStruct((_SEG_ROWS, 2 * _H), jnp.float32),
        mesh=plsc.VectorSubcoreMesh(core_axis_name="c", subcore_axis_name="s"),
        scratch_types=[
            pltpu.VMEM((2, _CHUNK), jnp.int32),
            pltpu.VMEM((2, _CHUNK), jnp.int32),
            pltpu.VMEM((_CHUNK, 2 * _H), jnp.float32),
            pltpu.VMEM((_CHUNK, 2 * _H), jnp.float32),
            pltpu.VMEM((_CHUNK, 2 * _H), jnp.float32),
            pltpu.VMEM((_CHUNK, 2 * _H), jnp.float32),
            pltpu.VMEM((_CHUNK // 2, 2 * _H), jnp.float32),
            pltpu.SemaphoreType.DMA,
            pltpu.SemaphoreType.DMA,
            pltpu.SemaphoreType.DMA,
            pltpu.SemaphoreType.DMA,
        ],
    )
    def _sc_gather(ab_hbm, eidx_hbm, out_hbm, idx_s, idx_d,
                   buf_a0, buf_b0, buf_a1, buf_b1, out_buf,
                   sem_a0, sem_b0, sem_a1, sem_b1):
        wid = lax.axis_index("s") * 2 + lax.axis_index("c")
        bufs = ((buf_a0, buf_b0, sem_a0, sem_b0),
                (buf_a1, buf_b1, sem_a1, sem_b1))

        half = _CHUNK // 2

        def fire(j, b):
            c = wid + _NW * j

            @pl.when(c < _SEG_CHUNKS)
            def _():
                base = (coff + c) * half
                pltpu.sync_copy(eidx_hbm.at[0, pl.ds(base, half)],
                                idx_s.at[b, pl.ds(0, half)])
                pltpu.sync_copy(eidx_hbm.at[0, pl.ds(_E2 + base, half)],
                                idx_s.at[b, pl.ds(half, half)])
                pltpu.sync_copy(eidx_hbm.at[1, pl.ds(base, half)],
                                idx_d.at[b, pl.ds(0, half)])
                pltpu.sync_copy(eidx_hbm.at[1, pl.ds(_E2 + base, half)],
                                idx_d.at[b, pl.ds(half, half)])
                buf_a, buf_b, sem_a, sem_b = bufs[b]
                pltpu.async_copy(ab_hbm.at[idx_s.at[b]], buf_a, sem_a)
                pltpu.async_copy(ab_hbm.at[idx_d.at[b]], buf_b, sem_b)

        def drain_compute(j, b):
            c = wid + _NW * j

            @pl.when(c < _SEG_CHUNKS)
            def _():
                buf_a, buf_b, sem_a, sem_b = bufs[b]
                pltpu.make_async_copy(ab_hbm.at[idx_s.at[b]], buf_a,
                                      sem_a).wait()
                pltpu.make_async_copy(ab_hbm.at[idx_d.at[b]], buf_b,
                                      sem_b).wait()
                fire(j + 1, 1 - b)

                def add_row(r, carry):
                    for l in range(_H // 16):
                        sl = pl.ds(l * 16, 16)
                        sb = pl.ds(_H + l * 16, 16)
                        out_buf[r, sl] = buf_a[r, sl] + buf_b[r, sb]
                        out_buf[r, sb] = buf_a[half + r, sl] + buf_b[half + r, sb]
                    return carry

                lax.fori_loop(0, half, add_row, 0)
                pltpu.sync_copy(out_buf,
                                out_hbm.at[pl.ds(c * half, half)])

        fire(0, 0)

        def body(j2, carry):
            drain_compute(2 * j2, 0)
            drain_compute(2 * j2 + 1, 1)
            return carry

        lax.fori_loop(0, (_SEG_ITER + 1) // 2, body, 0)

    return _sc_gather


_R2 = 3200


def _mlp_body(h_ref, e_ref, ew_ref, b1_ref, g1_ref, bt1_ref,
              w2_ref, b2_ref, g2_ref, bt2_ref,
              w3_ref, b3_ref, g3_ref, bt3_ref,
              w4_ref, b4_ref, savg_ref, sbc_ref, out_ref):
    savg = savg_ref[...]
    sbc = sbc_ref[...]

    def dot(a, b):
        return jnp.dot(a, b, preferred_element_type=jnp.float32)

    h = (h_ref[...]
         + lax.dot_general(e_ref[...], ew_ref[...], (((0,), (0,)), ((), ())),
                           preferred_element_type=jnp.float32)
         + b1_ref[...])

    def ln_tanh(v, g, bt):
        mu2 = dot(v, savg)
        d = v - dot(mu2, sbc)
        var2 = dot(d * d, savg)
        rstd2 = lax.rsqrt(var2 + 1e-5)
        return jnp.tanh(d * dot(rstd2, sbc) * g + bt)

    h = ln_tanh(h, g1_ref[...], bt1_ref[...])
    h = dot(h, w2_ref[...]) + b2_ref[...]
    h = ln_tanh(h, g2_ref[...], bt2_ref[...])
    h = dot(h, w3_ref[...]) + b3_ref[...]
    h = ln_tanh(h, g3_ref[...], bt3_ref[...])
    out_ref[...] = (lax.dot_general(w4_ref[...], h, (((0,), (1,)), ((), ())),
                                    preferred_element_type=jnp.float32)
                    + b4_ref[...])


def _mlp(hpre, e2, ew, b1, g1, bt1, w2, b2, g2, bt2, w3, b3, g3, bt3,
         w4, b4, savg, sbc):
    nblk = _SEG_ROWS // _R2
    full = lambda shape: pl.BlockSpec(shape, lambda j: (0, 0))
    vec = full((1, 2 * _H))
    return pl.pallas_call(
        _mlp_body,
        grid=(nblk,),
        in_specs=[
            pl.BlockSpec((_R2, 2 * _H), lambda j: (j, 0)),
            pl.BlockSpec((2, _R2), lambda j: (0, j)),
            full((2, 2 * _H)),
            vec, vec, vec,
            full((2 * _H, 2 * _H)), vec, vec, vec,
            full((2 * _H, 2 * _H)), vec, vec, vec,
            full((2 * _H, 2)), full((1, 1)),
            full((2 * _H, 2)), full((2, 2 * _H)),
        ],
        out_specs=pl.BlockSpec((2, _R2), lambda j: (0, j)),
        out_shape=jax.ShapeDtypeStruct((2, _SEG_ROWS), jnp.float32),
        compiler_params=pltpu.CompilerParams(
            dimension_semantics=("arbitrary",)),
    )(hpre, e2, ew, b1, g1, bt1, w2, b2, g2, bt2, w3, b3, g3, bt3, w4, b4,
      savg, sbc)


def _blockdiag2(w):
    a, b = w.shape
    z = jnp.zeros((a, b), w.dtype)
    return jnp.concatenate([jnp.concatenate([w, z], axis=1),
                            jnp.concatenate([z, w], axis=1)], axis=0)


def kernel(x, e, edge_index, W1, b1, W2, b2, W3, b3, W4, b4,
           g1, bt1, g2, bt2, g3, bt3):
    f32 = jnp.float32
    w1ab = jnp.concatenate([W1[:_D], W1[_D:2 * _D]], axis=1)
    ab_tab = _precompute(x, w1ab)

    w1c = W1[2 * _D]
    zeros_h = jnp.zeros((_H,), f32)
    ew = jnp.stack([jnp.concatenate([w1c, zeros_h]),
                    jnp.concatenate([zeros_h, w1c])])
    ones_h = jnp.ones((_H, 1), f32)
    zeros_col = jnp.zeros((_H, 1), f32)
    savg = jnp.concatenate(
        [jnp.concatenate([ones_h, zeros_col], axis=1),
         jnp.concatenate([zeros_col, ones_h], axis=1)], axis=0) / _H
    sbc = (savg.T > 0).astype(f32) * 1.0

    two = lambda v: jnp.tile(v.reshape(1, _H), (1, 2))
    e2 = e.reshape(2, _E2)
    outs = []
    for k in range(_NSEG):
        hpre_k = _make_sc_gather(k * _SEG_CHUNKS)(ab_tab, edge_index)
        e2_k = lax.slice(e2, (0, k * _SEG_ROWS), (2, (k + 1) * _SEG_ROWS))
        outs.append(_mlp(hpre_k, e2_k, ew,
                         two(b1), two(g1), two(bt1),
                         _blockdiag2(W2), two(b2), two(g2), two(bt2),
                         _blockdiag2(W3), two(b3), two(g3), two(bt3),
                         _blockdiag2(W4), b4.reshape(1, 1),
                         savg, sbc))
    return jnp.concatenate(outs, axis=1).reshape(_E)


# speedup vs baseline: 1.7495x; 1.4770x over previous
import functools

import jax
import jax.numpy as jnp
from jax import lax
from jax.experimental import pallas as pl
from jax.experimental.pallas import tpu as pltpu
from jax.experimental.pallas import tpu_sc as plsc

_N = 10000
_E = 320000
_D = 128
_H = 64
_E2 = _E // 2

_CHUNK = 128
_NW = 32
_NSEG = 10
_SEG_ROWS = _E2 // _NSEG
_SEG_CHUNKS = _SEG_ROWS // (_CHUNK // 2)
_SEG_ITER = -(-_SEG_CHUNKS // _NW)


def _precompute_body(x_ref, w_ref, ab_ref):
    ab_ref[...] = jnp.dot(x_ref[...], w_ref[...],
                          preferred_element_type=jnp.float32)


def _precompute(x, w1ab):
    return pl.pallas_call(
        _precompute_body,
        out_shape=jax.ShapeDtypeStruct((_N, 2 * _H), jnp.float32),
    )(x, w1ab)


@functools.cache
def _make_sc_gather(coff):
    @functools.partial(
        pl.kernel,
        out_type=jax.ShapeDtypeStruct((_SEG_ROWS, 2 * _H), jnp.float32),
        mesh=plsc.VectorSubcoreMesh(core_axis_name="c", subcore_axis_name="s"),
        scratch_types=[
            pltpu.VMEM((2, _CHUNK), jnp.int32),
            pltpu.VMEM((2, _CHUNK), jnp.int32),
            pltpu.VMEM((_CHUNK, 2 * _H), jnp.float32),
            pltpu.VMEM((_CHUNK, 2 * _H), jnp.float32),
            pltpu.VMEM((_CHUNK, 2 * _H), jnp.float32),
            pltpu.VMEM((_CHUNK, 2 * _H), jnp.float32),
            pltpu.VMEM((_CHUNK // 2, 2 * _H), jnp.float32),
            pltpu.SemaphoreType.DMA,
            pltpu.SemaphoreType.DMA,
            pltpu.SemaphoreType.DMA,
            pltpu.SemaphoreType.DMA,
            pltpu.SemaphoreType.DMA,
            pltpu.SemaphoreType.DMA,
        ],
    )
    def _sc_gather(ab_hbm, eidx_hbm, out_hbm, idx_s, idx_d,
                   buf_a0, buf_b0, buf_a1, buf_b1, out_buf,
                   sem_a0, sem_b0, sem_a1, sem_b1, sem_i0, sem_i1):
        wid = lax.axis_index("s") * 2 + lax.axis_index("c")
        bufs = ((buf_a0, buf_b0, sem_a0, sem_b0),
                (buf_a1, buf_b1, sem_a1, sem_b1))
        isems = (sem_i0, sem_i1)

        half = _CHUNK // 2

        def idx_copies(j, b):
            c = wid + _NW * j
            base = (coff + c) * half
            return (
                (eidx_hbm.at[0, pl.ds(base, half)],
                 idx_s.at[b, pl.ds(0, half)]),
                (eidx_hbm.at[0, pl.ds(_E2 + base, half)],
                 idx_s.at[b, pl.ds(half, half)]),
                (eidx_hbm.at[1, pl.ds(base, half)],
                 idx_d.at[b, pl.ds(0, half)]),
                (eidx_hbm.at[1, pl.ds(_E2 + base, half)],
                 idx_d.at[b, pl.ds(half, half)]),
            )

        def stage_idx(j, b):
            c = wid + _NW * j

            @pl.when(c < _SEG_CHUNKS)
            def _():
                for src, dst in idx_copies(j, b):
                    pltpu.async_copy(src, dst, isems[b])

        def wait_idx(j, b):
            for src, dst in idx_copies(j, b):
                pltpu.make_async_copy(src, dst, isems[b]).wait()

        def fire_gather(j, b):
            c = wid + _NW * j

            @pl.when(c < _SEG_CHUNKS)
            def _():
                wait_idx(j, b)
                buf_a, buf_b, sem_a, sem_b = bufs[b]
                pltpu.async_copy(ab_hbm.at[idx_s.at[b]], buf_a, sem_a)
                pltpu.async_copy(ab_hbm.at[idx_d.at[b]], buf_b, sem_b)

        def step(j, b):
            c = wid + _NW * j

            @pl.when(c < _SEG_CHUNKS)
            def _():
                buf_a, buf_b, sem_a, sem_b = bufs[b]
                pltpu.make_async_copy(ab_hbm.at[idx_s.at[b]], buf_a,
                                      sem_a).wait()
                pltpu.make_async_copy(ab_hbm.at[idx_d.at[b]], buf_b,
                                      sem_b).wait()
                fire_gather(j + 1, 1 - b)
                stage_idx(j + 2, b)

                def add_row(r, carry):
                    for l in range(_H // 16):
                        sl = pl.ds(l * 16, 16)
                        sb = pl.ds(_H + l * 16, 16)
                        out_buf[r, sl] = buf_a[r, sl] + buf_b[r, sb]
                        out_buf[r, sb] = buf_a[half + r, sl] + buf_b[half + r, sb]
                    return carry

                lax.fori_loop(0, half, add_row, 0)
                pltpu.sync_copy(out_buf,
                                out_hbm.at[pl.ds(c * half, half)])

        stage_idx(0, 0)
        fire_gather(0, 0)
        stage_idx(1, 1)

        def body(j2, carry):
            step(2 * j2, 0)
            step(2 * j2 + 1, 1)
            return carry

        lax.fori_loop(0, (_SEG_ITER + 1) // 2, body, 0)

    return _sc_gather


_R2 = 3200


def _mlp_body(h_ref, e_ref, ew_ref, b1_ref, g1_ref, bt1_ref,
              w2_ref, b2_ref, g2_ref, bt2_ref,
              w3_ref, b3_ref, g3_ref, bt3_ref,
              w4_ref, b4_ref, savg_ref, sbc_ref, out_ref):
    savg = savg_ref[...]
    sbc = sbc_ref[...]

    def dot(a, b):
        return jnp.dot(a, b, preferred_element_type=jnp.float32)

    h = (h_ref[...]
         + lax.dot_general(e_ref[...], ew_ref[...], (((0,), (0,)), ((), ())),
                           preferred_element_type=jnp.float32)
         + b1_ref[...])

    def ln_tanh(v, g, bt):
        mu2 = dot(v, savg)
        d = v - dot(mu2, sbc)
        var2 = dot(d * d, savg)
        rstd2 = lax.rsqrt(var2 + 1e-5)
        return jnp.tanh(d * dot(rstd2, sbc) * g + bt)

    h = ln_tanh(h, g1_ref[...], bt1_ref[...])
    h = dot(h, w2_ref[...]) + b2_ref[...]
    h = ln_tanh(h, g2_ref[...], bt2_ref[...])
    h = dot(h, w3_ref[...]) + b3_ref[...]
    h = ln_tanh(h, g3_ref[...], bt3_ref[...])
    out_ref[...] = (lax.dot_general(w4_ref[...], h, (((0,), (1,)), ((), ())),
                                    preferred_element_type=jnp.float32)
                    + b4_ref[...])


def _mlp(hpre, e2, ew, b1, g1, bt1, w2, b2, g2, bt2, w3, b3, g3, bt3,
         w4, b4, savg, sbc):
    nblk = _SEG_ROWS // _R2
    full = lambda shape: pl.BlockSpec(shape, lambda j: (0, 0))
    vec = full((1, 2 * _H))
    return pl.pallas_call(
        _mlp_body,
        grid=(nblk,),
        in_specs=[
            pl.BlockSpec((_R2, 2 * _H), lambda j: (j, 0)),
            pl.BlockSpec((2, _R2), lambda j: (0, j)),
            full((2, 2 * _H)),
            vec, vec, vec,
            full((2 * _H, 2 * _H)), vec, vec, vec,
            full((2 * _H, 2 * _H)), vec, vec, vec,
            full((2 * _H, 2)), full((1, 1)),
            full((2 * _H, 2)), full((2, 2 * _H)),
        ],
        out_specs=pl.BlockSpec((2, _R2), lambda j: (0, j)),
        out_shape=jax.ShapeDtypeStruct((2, _SEG_ROWS), jnp.float32),
        compiler_params=pltpu.CompilerParams(
            dimension_semantics=("arbitrary",)),
    )(hpre, e2, ew, b1, g1, bt1, w2, b2, g2, bt2, w3, b3, g3, bt3, w4, b4,
      savg, sbc)


def _blockdiag2(w):
    a, b = w.shape
    z = jnp.zeros((a, b), w.dtype)
    return jnp.concatenate([jnp.concatenate([w, z], axis=1),
                            jnp.concatenate([z, w], axis=1)], axis=0)


def kernel(x, e, edge_index, W1, b1, W2, b2, W3, b3, W4, b4,
           g1, bt1, g2, bt2, g3, bt3):
    f32 = jnp.float32
    w1ab = jnp.concatenate([W1[:_D], W1[_D:2 * _D]], axis=1)
    ab_tab = _precompute(x, w1ab)

    w1c = W1[2 * _D]
    zeros_h = jnp.zeros((_H,), f32)
    ew = jnp.stack([jnp.concatenate([w1c, zeros_h]),
                    jnp.concatenate([zeros_h, w1c])])
    ones_h = jnp.ones((_H, 1), f32)
    zeros_col = jnp.zeros((_H, 1), f32)
    savg = jnp.concatenate(
        [jnp.concatenate([ones_h, zeros_col], axis=1),
         jnp.concatenate([zeros_col, ones_h], axis=1)], axis=0) / _H
    sbc = (savg.T > 0).astype(f32) * 1.0

    two = lambda v: jnp.tile(v.reshape(1, _H), (1, 2))
    e2 = e.reshape(2, _E2)
    outs = []
    for k in range(_NSEG):
        hpre_k = _make_sc_gather(k * _SEG_CHUNKS)(ab_tab, edge_index)
        e2_k = lax.slice(e2, (0, k * _SEG_ROWS), (2, (k + 1) * _SEG_ROWS))
        outs.append(_mlp(hpre_k, e2_k, ew,
                         two(b1), two(g1), two(bt1),
                         _blockdiag2(W2), two(b2), two(g2), two(bt2),
                         _blockdiag2(W3), two(b3), two(g3), two(bt3),
                         _blockdiag2(W4), b4.reshape(1, 1),
                         savg, sbc))
    return jnp.concatenate(outs, axis=1).reshape(_E)
